# baseline (speedup 1.0000x reference)
import functools

import jax
import jax.numpy as jnp
from jax import lax
from jax.experimental import pallas as pl
from jax.experimental.pallas import tpu as pltpu
from jax.experimental.pallas import tpu_sc as plsc

N_NODES = 10000
N_EDGES = 160000
G = 256
X_IN = 128
E_DIM = 16
HID = 512
NP1 = 10
FPW = 5120
FP_IN = 2513

NPAD = 10240
NBLK = 256
NBLOCKS = NPAD // NBLK
EPAD = 163840
ECHUNK = 128
ECHUNKE = 64
FP1_OUT = 6784
FP1_REAL = 6656


def _edge_agg(x_pad, ea_pad, sd2, dst_pad):
    info = plsc.get_sparse_core_info()
    NC, NS = info.num_cores, info.num_subcores
    epw = EPAD // NS
    nchunks = epw // ECHUNK
    rows_per_tile = NPAD // NS
    mesh = plsc.VectorSubcoreMesh(core_axis_name="c", subcore_axis_name="s")

    @functools.partial(
        pl.kernel, mesh=mesh,
        out_type=jax.ShapeDtypeStruct((NC, NPAD, X_IN), jnp.float32),
        scratch_types=[
            pltpu.VMEM((2, 2, ECHUNK), jnp.int32),
            pltpu.VMEM((2, ECHUNK, X_IN), jnp.float32),
            pltpu.VMEM((ECHUNKE, E_DIM), jnp.float32),
            pltpu.VMEM((ECHUNKE,), jnp.int32),
            pltpu.VMEM((32, X_IN), jnp.float32),
            pltpu.VMEM_SHARED((NPAD, X_IN), jnp.float32),
            pltpu.SemaphoreType.DMA,
            pltpu.SemaphoreType.DMA,
        ],
    )
    def k(x_hbm, ea_hbm, sd_hbm, dst_hbm, out,
          sdx, rows, erows, didxe, rbx, acc, sema, semb):
        cid = lax.axis_index("c")
        sid = lax.axis_index("s")
        sems = (sema, semb)

        zvec = jnp.zeros((16,), jnp.float32)

        def zrow(i, _):
            for j in range(X_IN // 16):
                rbx[i, pl.ds(j * 16, 16)] = zvec
            return 0
        lax.fori_loop(0, 32, zrow, 0)

        tile_base = sid * rows_per_tile

        def zcopy(t, _):
            pltpu.sync_copy(rbx, acc.at[pl.ds(tile_base + t * 32, 32)])
            return 0
        lax.fori_loop(0, rows_per_tile // 32, zcopy, 0)
        plsc.subcore_barrier()

        ebase = sid * epw

        @pl.when(cid == 0)
        def _xpath():
            crow = sid * nchunks * 2
            pltpu.sync_copy(sd_hbm.at[pl.ds(crow, 2)], sdx.at[0])
            pltpu.async_copy(x_hbm.at[sdx.at[0, 0]], rows.at[0], sema)

            def body2(i2, _):
                for b in range(2):
                    g = i2 * 2 + b
                    nb = 1 - b

                    @pl.when(g + 1 < nchunks)
                    def _prefetch():
                        pltpu.sync_copy(
                            sd_hbm.at[pl.ds(crow + (g + 1) * 2, 2)],
                            sdx.at[nb])
                        pltpu.async_copy(x_hbm.at[sdx.at[nb, 0]], rows.at[nb],
                                         sems[nb])
                    pltpu.make_async_copy(x_hbm.at[sdx.at[b, 0]], rows.at[b],
                                          sems[b]).wait()
                    pltpu.sync_copy(rows.at[b], acc.at[sdx.at[b, 1]], add=True)
                return 0
            lax.fori_loop(0, nchunks // 2, body2, 0)

        @pl.when(cid == 1)
        def _epath():
            def zr(i, _):
                for j in range(X_IN // 16):
                    rows[0, i, pl.ds(j * 16, 16)] = zvec
                return 0
            lax.fori_loop(0, ECHUNKE, zr, 0)

            def chunk(g, _):
                off = ebase + g * ECHUNKE
                pltpu.sync_copy(dst_hbm.at[pl.ds(off, ECHUNKE)], didxe)
                pltpu.sync_copy(ea_hbm.at[pl.ds(off, ECHUNKE)], erows)

                def cp(r, _):
                    rows[0, r, pl.ds(0, 16)] = erows[r, :]
                    return 0
                lax.fori_loop(0, ECHUNKE, cp, 0)
                pltpu.sync_copy(rows.at[0, pl.ds(0, ECHUNKE)],
                                acc.at[didxe], add=True)
                return 0
            lax.fori_loop(0, epw // ECHUNKE, chunk, 0)

        plsc.subcore_barrier()

        def rb(t, _):
            r0 = tile_base + t * 32
            pltpu.sync_copy(acc.at[pl.ds(r0, 32)], rbx)
            pltpu.sync_copy(rbx, out.at[cid, pl.ds(r0, 32)])
            return 0
        lax.fori_loop(0, rows_per_tile // 32, rb, 0)

    return k(x_pad, ea_pad, sd2, dst_pad)


def _dense_segmax_body(x_ref, xa_ref, ea_ref, bf_ref,
                       wn_ref, ws_ref, we_ref, b_ref, feats_ref, gp_ref):
    i = pl.program_id(0)

    @pl.when(i == 0)
    def _init():
        feats_ref[...] = jnp.full((G, FPW), -jnp.inf, jnp.float32)
        gp_ref[...] = jnp.full((G, X_IN), -jnp.inf, jnp.float32)

    xb = x_ref[...]
    xagg = xa_ref[...]
    eagg = ea_ref[...][:, 0:E_DIM]
    pre = (jnp.dot(xagg, wn_ref[...], preferred_element_type=jnp.float32)
           + jnp.dot(xb, ws_ref[...], preferred_element_type=jnp.float32)
           + jnp.dot(eagg, we_ref[...], preferred_element_type=jnp.float32)
           + b_ref[...])
    h = jnp.maximum(pre, 0.0)
    bcol = bf_ref[:, 0:1]
    valid = bcol < 256.0
    h = jnp.where(valid, h, -jnp.inf)
    xm = jnp.where(valid, xb, -jnp.inf)

    gmin = bf_ref[0, 0]
    gmax = jnp.minimum(bf_ref[NBLK - 1, 0], 255.0)
    n = (gmax - gmin).astype(jnp.int32) + 1

    def body(t, _):
        gf = gmin + t.astype(jnp.float32)
        gi = gf.astype(jnp.int32)
        m = bcol == gf
        hm = jnp.max(jnp.where(m, h, -jnp.inf), axis=0, keepdims=True)
        xmm = jnp.max(jnp.where(m, xm, -jnp.inf), axis=0, keepdims=True)
        feats_ref[pl.ds(gi, 1), :] = jnp.maximum(feats_ref[pl.ds(gi, 1), :], hm)
        gp_ref[pl.ds(gi, 1), :] = jnp.maximum(gp_ref[pl.ds(gi, 1), :], xmm)
        return 0
    lax.fori_loop(0, n, body, 0)


def _dense_segmax(x_pad, xa, ea, batchf, wn, ws, we, b):
    blk = lambda r, c: pl.BlockSpec((r, c), lambda i: (i, 0))
    whole = lambda r, c: pl.BlockSpec((r, c), lambda i: (0, 0))
    return pl.pallas_call(
        _dense_segmax_body,
        grid=(NBLOCKS,),
        in_specs=[blk(NBLK, X_IN), blk(NBLK, X_IN), blk(NBLK, X_IN),
                  blk(NBLK, X_IN),
                  whole(X_IN, FPW), whole(X_IN, FPW), whole(E_DIM, FPW),
                  whole(1, FPW)],
        out_specs=[whole(G, FPW), whole(G, X_IN)],
        out_shape=[jax.ShapeDtypeStruct((G, FPW), jnp.float32),
                   jax.ShapeDtypeStruct((G, X_IN), jnp.float32)],
    )(x_pad, xa, ea, batchf, wn, ws, we, b)


def _fp1a_body(fpx_ref, w0, w1, w2, w3, w4, o_ref):
    x = fpx_ref[...]
    offs = (0, 512, 1024, 1536, 2560)
    for off, wref in zip(offs, (w0, w1, w2, w3, w4)):
        wdt = wref.shape[1]
        o_ref[:, off:off + wdt] = jnp.dot(x, wref[...],
                                          preferred_element_type=jnp.float32)


def _fp1b_body(fpx_ref, w5, w6, wg2_ref, o_ref):
    x = fpx_ref[...]
    for off, wref in zip((0, 1536, 3072), (w5, w6, wg2_ref)):
        wdt = wref.shape[1]
        o_ref[:, off:off + wdt] = jnp.dot(x, wref[...],
                                          preferred_element_type=jnp.float32)


def _fp1(fp_x, firsts, wg2p):
    h1a = pl.pallas_call(
        _fp1a_body,
        out_shape=jax.ShapeDtypeStruct((G, 3584), jnp.float32),
    )(fp_x, *firsts[:5])
    h1b = pl.pallas_call(
        _fp1b_body,
        out_shape=jax.ShapeDtypeStruct((G, 3200), jnp.float32),
    )(fp_x, firsts[5], firsts[6], wg2p)
    return h1a, h1b


def _combine_body(h1a_ref, h1b_ref, feats_ref, gp_ref, wg1_ref, bg1_ref, bg2_ref,
                  e1w2_ref, e2w2_ref, e2w3_ref, e3w2_ref, e4w2_ref, e4w3_ref,
                  e5w2_ref, e6w2_ref, e6w3_ref,
                  watt_ref, wp1_ref, bp1_ref, gamma_ref, beta_ref, wp2_ref,
                  scal_ref, o_ref):
    f32 = jnp.float32
    dot = lambda a, b: jnp.dot(a, b, preferred_element_type=f32)
    relu = lambda v: jnp.maximum(v, 0.0)
    h1a = h1a_ref[...]
    h1b = h1b_ref[...]

    logits1 = dot(gp_ref[...], wg1_ref[...]) + bg1_ref[...]
    col = lax.broadcasted_iota(jnp.int32, (G, 128), 1)
    logits1 = jnp.where(col < NP1, logits1, -jnp.inf)
    w1 = jax.nn.softmax(logits1, axis=1)

    feats = feats_ref[...]
    out1 = jnp.zeros((G, HID), f32)
    for i in range(NP1):
        out1 = out1 + w1[:, i:i + 1] * feats[:, i * HID:(i + 1) * HID]
    n1 = jnp.maximum(jnp.sqrt(jnp.sum(out1 * out1, axis=1, keepdims=True)), 1e-12)
    out1 = out1 / n1

    logits2 = h1b[:, 3072:3200] + bg2_ref[...]
    logits2 = jnp.where(col < 7, logits2, -jnp.inf)
    w2 = jax.nn.softmax(logits2, axis=1)

    f0 = h1a[:, 0:512]
    f1 = dot(relu(h1a[:, 512:1024]), e1w2_ref[...])
    f2 = dot(relu(dot(relu(h1a[:, 1024:1536]), e2w2_ref[...])), e2w3_ref[...])
    f3 = dot(relu(h1a[:, 1536:2560]), e3w2_ref[...])
    f4 = dot(relu(dot(relu(h1a[:, 2560:3584]), e4w2_ref[...])), e4w3_ref[...])
    f5 = dot(relu(h1b[:, 0:1536]), e5w2_ref[...])
    f6 = dot(relu(dot(relu(h1b[:, 1536:3072]), e6w2_ref[...])), e6w3_ref[...])
    fs = (f0, f1, f2, f3, f4, f5, f6)
    out2 = jnp.zeros((G, HID), f32)
    for i in range(7):
        out2 = out2 + w2[:, i:i + 1] * fs[i]
    n2 = jnp.maximum(jnp.sqrt(jnp.sum(out2 * out2, axis=1, keepdims=True)), 1e-12)
    out2 = out2 / n2

    watt = watt_ref[...]
    a = (jnp.sum(out1 * watt[:, 0:HID], axis=1, keepdims=True)
         + jnp.sum(out2 * watt[:, HID:2 * HID], axis=1, keepdims=True)
         + scal_ref[0, 0])
    alpha = 1.0 / (1.0 + jnp.exp(-a))
    mix = alpha * out1 + (1.0 - alpha) * out2

    hh = relu(dot(mix, wp1_ref[...]) + bp1_ref[...])
    mu = jnp.mean(hh, axis=0, keepdims=True)
    var = jnp.mean((hh - mu) * (hh - mu), axis=0, keepdims=True)
    hh = (hh - mu) / jnp.sqrt(var + 1e-5) * gamma_ref[...] + beta_ref[...]
    res = jnp.sum(hh * wp2_ref[...], axis=1, keepdims=True) + scal_ref[0, 1]
    o_ref[...] = jnp.broadcast_to(res, (G, 128))


def _combine(h1a, h1b, feats, gpool, wg1p, bg1p, bg2p, tails, wattT, wp1, bp1,
             gamma2, beta2, wp2T, scal):
    return pl.pallas_call(
        _combine_body,
        out_shape=jax.ShapeDtypeStruct((G, 128), jnp.float32),
    )(h1a, h1b, feats, gpool, wg1p, bg1p, bg2p, *tails, wattT, wp1, bp1,
      gamma2, beta2, wp2T, scal)


def kernel(x, edge_index, batch, y, edge_attr, w, pubchem, maccs, erg, ecfp,
           params):
    f32 = jnp.float32
    x_pad = jnp.zeros((NPAD, X_IN), f32).at[:N_NODES].set(x)
    batchf = jnp.full((NPAD,), 300.0, f32).at[:N_NODES].set(batch.astype(f32))
    batchf = jnp.broadcast_to(batchf[:, None], (NPAD, X_IN))

    src = edge_index[0].astype(jnp.int32)
    dst = edge_index[1].astype(jnp.int32)
    src_pad = jnp.full((EPAD,), NPAD - 1, jnp.int32).at[:N_EDGES].set(src)
    dst_pad = jnp.zeros((EPAD,), jnp.int32).at[:N_EDGES].set(dst)
    ea_pad = jnp.zeros((EPAD, E_DIM), f32).at[:N_EDGES].set(edge_attr)

    sd2 = jnp.stack([src_pad.reshape(EPAD // ECHUNK, ECHUNK),
                     dst_pad.reshape(EPAD // ECHUNK, ECHUNK)], axis=1)
    sd2 = sd2.reshape(2 * (EPAD // ECHUNK), ECHUNK)

    agg = _edge_agg(x_pad, ea_pad, sd2, dst_pad)

    wn = params["Wn"].transpose(1, 0, 2).reshape(X_IN, FPW)
    ws = params["Wself"].transpose(1, 0, 2).reshape(X_IN, FPW)
    we = params["We"].transpose(1, 0, 2).reshape(E_DIM, FPW)
    b = params["bexp"].reshape(1, FPW)

    feats, gpool = _dense_segmax(x_pad, agg[0], agg[1], batchf, wn, ws, we, b)

    fp_x = jnp.concatenate([pubchem, maccs, erg, ecfp], axis=1)
    firsts = [layers[0] for layers in params["fp"]]
    wg2p = jnp.zeros((FP_IN, 128), f32).at[:, :7].set(params["Wg2"])
    h1a, h1b = _fp1(fp_x, firsts, wg2p)

    fp = params["fp"]
    tails = (fp[1][1], fp[2][1], fp[2][2], fp[3][1], fp[4][1], fp[4][2],
             fp[5][1], fp[6][1], fp[6][2])
    wg1p = jnp.zeros((X_IN, 128), f32).at[:, :NP1].set(params["Wg1"])
    bg1p = jnp.zeros((1, 128), f32).at[0, :NP1].set(params["bg1"])
    bg2p = jnp.zeros((1, 128), f32).at[0, :7].set(params["bg2"])
    wattT = params["Watt"].reshape(1, 2 * HID)
    scal = jnp.zeros((1, 128), f32)
    scal = scal.at[0, 0].set(params["batt"][0]).at[0, 1].set(params["bp2"][0])
    outc = _combine(h1a, h1b, feats, gpool, wg1p, bg1p, bg2p, tails, wattT,
                    params["Wp1"], params["bp1"].reshape(1, HID),
                    params["gamma"].reshape(1, HID),
                    params["beta"].reshape(1, HID),
                    params["Wp2"].reshape(1, HID), scal)
    out = outc[:, 0]

    mask = y != 999.0
    out_m = jnp.where(mask, out, 0.0)
    y_m = jnp.where(mask, y, 0.0)
    w_m = jnp.where(mask, w, 0.0)
    return (out, y, w, out_m, y_m, w_m)


# speedup vs baseline: 1.0155x; 1.0155x over previous
import functools

import jax
import jax.numpy as jnp
from jax import lax
from jax.experimental import pallas as pl
from jax.experimental.pallas import tpu as pltpu
from jax.experimental.pallas import tpu_sc as plsc

N_NODES = 10000
N_EDGES = 160000
G = 256
X_IN = 128
E_DIM = 16
HID = 512
NP1 = 10
FPW = 5120
FP_IN = 2513

NPAD = 10240
NBLK = 256
NBLOCKS = NPAD // NBLK
EPAD = 163840
ECHUNK = 128
ECHUNKE = 64
FP1_OUT = 6784
FP1_REAL = 6656


def _edge_agg(x_pad, ea_pad, src_pad, dst_pad):
    info = plsc.get_sparse_core_info()
    NC, NS = info.num_cores, info.num_subcores
    epw = EPAD // NS
    nchunks = epw // ECHUNK
    rows_per_tile = NPAD // NS
    mesh = plsc.VectorSubcoreMesh(core_axis_name="c", subcore_axis_name="s")

    @functools.partial(
        pl.kernel, mesh=mesh,
        out_type=jax.ShapeDtypeStruct((NC, NPAD, X_IN), jnp.float32),
        scratch_types=[
            pltpu.VMEM((2, ECHUNK), jnp.int32),
            pltpu.VMEM((2, ECHUNK), jnp.int32),
            pltpu.VMEM((2, ECHUNK, X_IN), jnp.float32),
            pltpu.VMEM((ECHUNKE, E_DIM), jnp.float32),
            pltpu.VMEM((ECHUNKE,), jnp.int32),
            pltpu.VMEM((32, X_IN), jnp.float32),
            pltpu.VMEM_SHARED((NPAD, X_IN), jnp.float32),
            pltpu.SemaphoreType.DMA,
            pltpu.SemaphoreType.DMA,
        ],
    )
    def k(x_hbm, ea_hbm, src_hbm, dst_hbm, out,
          sidx, didx, rows, erows, didxe, rbx, acc, sema, semb):
        cid = lax.axis_index("c")
        sid = lax.axis_index("s")
        sems = (sema, semb)

        zvec = jnp.zeros((16,), jnp.float32)

        def zrow(i, _):
            for j in range(X_IN // 16):
                rbx[i, pl.ds(j * 16, 16)] = zvec
            return 0
        lax.fori_loop(0, 32, zrow, 0)

        tile_base = sid * rows_per_tile

        def zcopy(t, _):
            pltpu.sync_copy(rbx, acc.at[pl.ds(tile_base + t * 32, 32)])
            return 0
        lax.fori_loop(0, rows_per_tile // 32, zcopy, 0)
        plsc.subcore_barrier()

        ebase = sid * epw

        @pl.when(cid == 0)
        def _xpath():
            pltpu.sync_copy(src_hbm.at[pl.ds(ebase, ECHUNK)], sidx.at[0])
            pltpu.sync_copy(dst_hbm.at[pl.ds(ebase, ECHUNK)], didx.at[0])
            pltpu.async_copy(x_hbm.at[sidx.at[0]], rows.at[0], sema)

            def body2(i2, _):
                for b in range(2):
                    g = i2 * 2 + b
                    nb = 1 - b

                    @pl.when(g + 1 < nchunks)
                    def _prefetch():
                        noff = ebase + (g + 1) * ECHUNK
                        pltpu.sync_copy(src_hbm.at[pl.ds(noff, ECHUNK)],
                                        sidx.at[nb])
                        pltpu.sync_copy(dst_hbm.at[pl.ds(noff, ECHUNK)],
                                        didx.at[nb])
                        pltpu.async_copy(x_hbm.at[sidx.at[nb]], rows.at[nb],
                                         sems[nb])
                    pltpu.make_async_copy(x_hbm.at[sidx.at[b]], rows.at[b],
                                          sems[b]).wait()
                    pltpu.sync_copy(rows.at[b], acc.at[didx.at[b]], add=True)
                return 0
            lax.fori_loop(0, nchunks // 2, body2, 0)

        @pl.when(cid == 1)
        def _epath():
            def zr(i, _):
                for j in range(X_IN // 16):
                    rows[0, i, pl.ds(j * 16, 16)] = zvec
                return 0
            lax.fori_loop(0, ECHUNKE, zr, 0)

            def chunk(g, _):
                off = ebase + g * ECHUNKE
                pltpu.sync_copy(dst_hbm.at[pl.ds(off, ECHUNKE)], didxe)
                pltpu.sync_copy(ea_hbm.at[pl.ds(off, ECHUNKE)], erows)

                def cp(r, _):
                    rows[0, r, pl.ds(0, 16)] = erows[r, :]
                    return 0
                lax.fori_loop(0, ECHUNKE, cp, 0)
                pltpu.sync_copy(rows.at[0, pl.ds(0, ECHUNKE)],
                                acc.at[didxe], add=True)
                return 0
            lax.fori_loop(0, epw // ECHUNKE, chunk, 0)

        plsc.subcore_barrier()

        def rb(t, _):
            r0 = tile_base + t * 32
            pltpu.sync_copy(acc.at[pl.ds(r0, 32)], rbx)
            pltpu.sync_copy(rbx, out.at[cid, pl.ds(r0, 32)])
            return 0
        lax.fori_loop(0, rows_per_tile // 32, rb, 0)

    return k(x_pad, ea_pad, src_pad, dst_pad)


def _dense_segmax_body(x_ref, xa_ref, ea_ref, bf_ref,
                       wn_ref, ws_ref, we_ref, b_ref, feats_ref, gp_ref):
    i = pl.program_id(0)

    @pl.when(i == 0)
    def _init():
        feats_ref[...] = jnp.full((G, FPW), -jnp.inf, jnp.float32)
        gp_ref[...] = jnp.full((G, X_IN), -jnp.inf, jnp.float32)

    xb = x_ref[...]
    xagg = xa_ref[...]
    eagg = ea_ref[...][:, 0:E_DIM]
    pre = (jnp.dot(xagg, wn_ref[...], preferred_element_type=jnp.float32)
           + jnp.dot(xb, ws_ref[...], preferred_element_type=jnp.float32)
           + jnp.dot(eagg, we_ref[...], preferred_element_type=jnp.float32)
           + b_ref[...])
    h = jnp.maximum(pre, 0.0)
    bcol = bf_ref[:, 0:1]
    valid = bcol < 256.0
    h = jnp.where(valid, h, -jnp.inf)
    xm = jnp.where(valid, xb, -jnp.inf)

    gmin = bf_ref[0, 0]
    gmax = jnp.minimum(bf_ref[NBLK - 1, 0], 255.0)
    n = (gmax - gmin).astype(jnp.int32) + 1

    def body(t, _):
        gf = gmin + t.astype(jnp.float32)
        gi = gf.astype(jnp.int32)
        m = bcol == gf
        hm = jnp.max(jnp.where(m, h, -jnp.inf), axis=0, keepdims=True)
        xmm = jnp.max(jnp.where(m, xm, -jnp.inf), axis=0, keepdims=True)
        feats_ref[pl.ds(gi, 1), :] = jnp.maximum(feats_ref[pl.ds(gi, 1), :], hm)
        gp_ref[pl.ds(gi, 1), :] = jnp.maximum(gp_ref[pl.ds(gi, 1), :], xmm)
        return 0
    lax.fori_loop(0, n, body, 0)


def _dense_segmax(x_pad, xa, ea, batchf, wn, ws, we, b):
    blk = lambda r, c: pl.BlockSpec((r, c), lambda i: (i, 0))
    whole = lambda r, c: pl.BlockSpec((r, c), lambda i: (0, 0))
    return pl.pallas_call(
        _dense_segmax_body,
        grid=(NBLOCKS,),
        in_specs=[blk(NBLK, X_IN), blk(NBLK, X_IN), blk(NBLK, X_IN),
                  blk(NBLK, X_IN),
                  whole(X_IN, FPW), whole(X_IN, FPW), whole(E_DIM, FPW),
                  whole(1, FPW)],
        out_specs=[whole(G, FPW), whole(G, X_IN)],
        out_shape=[jax.ShapeDtypeStruct((G, FPW), jnp.float32),
                   jax.ShapeDtypeStruct((G, X_IN), jnp.float32)],
    )(x_pad, xa, ea, batchf, wn, ws, we, b)


def _fp1a_body(fpx_ref, w0, w1, w2, w3, w4, o_ref):
    x = fpx_ref[...]
    offs = (0, 512, 1024, 1536, 2560)
    for off, wref in zip(offs, (w0, w1, w2, w3, w4)):
        wdt = wref.shape[1]
        o_ref[:, off:off + wdt] = jnp.dot(x, wref[...],
                                          preferred_element_type=jnp.float32)


def _fp1b_body(fpx_ref, w5, w6, wg2_ref, o_ref):
    x = fpx_ref[...]
    for off, wref in zip((0, 1536, 3072), (w5, w6, wg2_ref)):
        wdt = wref.shape[1]
        o_ref[:, off:off + wdt] = jnp.dot(x, wref[...],
                                          preferred_element_type=jnp.float32)


def _fp1(fp_x, firsts, wg2p):
    h1a = pl.pallas_call(
        _fp1a_body,
        out_shape=jax.ShapeDtypeStruct((G, 3584), jnp.float32),
    )(fp_x, *firsts[:5])
    h1b = pl.pallas_call(
        _fp1b_body,
        out_shape=jax.ShapeDtypeStruct((G, 3200), jnp.float32),
    )(fp_x, firsts[5], firsts[6], wg2p)
    return h1a, h1b


def _combine_body(h1a_ref, h1b_ref, feats_ref, gp_ref, wg1_ref, bg1_ref, bg2_ref,
                  e1w2_ref, e2w2_ref, e2w3_ref, e3w2_ref, e4w2_ref, e4w3_ref,
                  e5w2_ref, e6w2_ref, e6w3_ref,
                  watt_ref, wp1_ref, bp1_ref, gamma_ref, beta_ref, wp2_ref,
                  scal_ref, o_ref):
    f32 = jnp.float32
    dot = lambda a, b: jnp.dot(a, b, preferred_element_type=f32)
    relu = lambda v: jnp.maximum(v, 0.0)
    h1a = h1a_ref[...]
    h1b = h1b_ref[...]

    logits1 = dot(gp_ref[...], wg1_ref[...]) + bg1_ref[...]
    col = lax.broadcasted_iota(jnp.int32, (G, 128), 1)
    logits1 = jnp.where(col < NP1, logits1, -jnp.inf)
    w1 = jax.nn.softmax(logits1, axis=1)

    feats = feats_ref[...]
    out1 = jnp.zeros((G, HID), f32)
    for i in range(NP1):
        out1 = out1 + w1[:, i:i + 1] * feats[:, i * HID:(i + 1) * HID]
    n1 = jnp.maximum(jnp.sqrt(jnp.sum(out1 * out1, axis=1, keepdims=True)), 1e-12)
    out1 = out1 / n1

    logits2 = h1b[:, 3072:3200] + bg2_ref[...]
    logits2 = jnp.where(col < 7, logits2, -jnp.inf)
    w2 = jax.nn.softmax(logits2, axis=1)

    f0 = h1a[:, 0:512]
    f1 = dot(relu(h1a[:, 512:1024]), e1w2_ref[...])
    f2 = dot(relu(dot(relu(h1a[:, 1024:1536]), e2w2_ref[...])), e2w3_ref[...])
    f3 = dot(relu(h1a[:, 1536:2560]), e3w2_ref[...])
    f4 = dot(relu(dot(relu(h1a[:, 2560:3584]), e4w2_ref[...])), e4w3_ref[...])
    f5 = dot(relu(h1b[:, 0:1536]), e5w2_ref[...])
    f6 = dot(relu(dot(relu(h1b[:, 1536:3072]), e6w2_ref[...])), e6w3_ref[...])
    fs = (f0, f1, f2, f3, f4, f5, f6)
    out2 = jnp.zeros((G, HID), f32)
    for i in range(7):
        out2 = out2 + w2[:, i:i + 1] * fs[i]
    n2 = jnp.maximum(jnp.sqrt(jnp.sum(out2 * out2, axis=1, keepdims=True)), 1e-12)
    out2 = out2 / n2

    watt = watt_ref[...]
    a = (jnp.sum(out1 * watt[:, 0:HID], axis=1, keepdims=True)
         + jnp.sum(out2 * watt[:, HID:2 * HID], axis=1, keepdims=True)
         + scal_ref[0, 0])
    alpha = 1.0 / (1.0 + jnp.exp(-a))
    mix = alpha * out1 + (1.0 - alpha) * out2

    hh = relu(dot(mix, wp1_ref[...]) + bp1_ref[...])
    mu = jnp.mean(hh, axis=0, keepdims=True)
    var = jnp.mean((hh - mu) * (hh - mu), axis=0, keepdims=True)
    hh = (hh - mu) / jnp.sqrt(var + 1e-5) * gamma_ref[...] + beta_ref[...]
    res = jnp.sum(hh * wp2_ref[...], axis=1, keepdims=True) + scal_ref[0, 1]
    o_ref[...] = jnp.broadcast_to(res, (G, 128))


def _combine(h1a, h1b, feats, gpool, wg1p, bg1p, bg2p, tails, wattT, wp1, bp1,
             gamma2, beta2, wp2T, scal):
    return pl.pallas_call(
        _combine_body,
        out_shape=jax.ShapeDtypeStruct((G, 128), jnp.float32),
    )(h1a, h1b, feats, gpool, wg1p, bg1p, bg2p, *tails, wattT, wp1, bp1,
      gamma2, beta2, wp2T, scal)


def kernel(x, edge_index, batch, y, edge_attr, w, pubchem, maccs, erg, ecfp,
           params):
    f32 = jnp.float32
    x_pad = jnp.zeros((NPAD, X_IN), f32).at[:N_NODES].set(x)
    batchf = jnp.full((NPAD,), 300.0, f32).at[:N_NODES].set(batch.astype(f32))
    batchf = jnp.broadcast_to(batchf[:, None], (NPAD, X_IN))

    src = edge_index[0].astype(jnp.int32)
    dst = edge_index[1].astype(jnp.int32)
    src_pad = jnp.full((EPAD,), NPAD - 1, jnp.int32).at[:N_EDGES].set(src)
    dst_pad = jnp.zeros((EPAD,), jnp.int32).at[:N_EDGES].set(dst)
    ea_pad = jnp.zeros((EPAD, E_DIM), f32).at[:N_EDGES].set(edge_attr)

    agg = _edge_agg(x_pad, ea_pad, src_pad, dst_pad)

    wn = params["Wn"].transpose(1, 0, 2).reshape(X_IN, FPW)
    ws = params["Wself"].transpose(1, 0, 2).reshape(X_IN, FPW)
    we = params["We"].transpose(1, 0, 2).reshape(E_DIM, FPW)
    b = params["bexp"].reshape(1, FPW)

    feats, gpool = _dense_segmax(x_pad, agg[0], agg[1], batchf, wn, ws, we, b)

    fp_x = jnp.concatenate([pubchem, maccs, erg, ecfp], axis=1)
    firsts = [layers[0] for layers in params["fp"]]
    wg2p = jnp.zeros((FP_IN, 128), f32).at[:, :7].set(params["Wg2"])
    h1a, h1b = _fp1(fp_x, firsts, wg2p)

    fp = params["fp"]
    tails = (fp[1][1], fp[2][1], fp[2][2], fp[3][1], fp[4][1], fp[4][2],
             fp[5][1], fp[6][1], fp[6][2])
    wg1p = jnp.zeros((X_IN, 128), f32).at[:, :NP1].set(params["Wg1"])
    bg1p = jnp.zeros((1, 128), f32).at[0, :NP1].set(params["bg1"])
    bg2p = jnp.zeros((1, 128), f32).at[0, :7].set(params["bg2"])
    wattT = params["Watt"].reshape(1, 2 * HID)
    scal = jnp.zeros((1, 128), f32)
    scal = scal.at[0, 0].set(params["batt"][0]).at[0, 1].set(params["bp2"][0])
    outc = _combine(h1a, h1b, feats, gpool, wg1p, bg1p, bg2p, tails, wattT,
                    params["Wp1"], params["bp1"].reshape(1, HID),
                    params["gamma"].reshape(1, HID),
                    params["beta"].reshape(1, HID),
                    params["Wp2"].reshape(1, HID), scal)
    out = outc[:, 0]

    mask = y != 999.0
    out_m = jnp.where(mask, out, 0.0)
    y_m = jnp.where(mask, y, 0.0)
    w_m = jnp.where(mask, w, 0.0)
    return (out, y, w, out_m, y_m, w_m)
